# baseline (speedup 1.0000x reference)
import functools
import math

import jax
import jax.numpy as jnp
import numpy as np
from jax import lax
from jax.experimental import pallas as pl
from jax.experimental.pallas import tpu as pltpu
from jax.experimental.pallas import tpu_sc as plsc

_NUM_CORES = 2
_NUM_SUBCORES = 16
_LANES = 16


def _positional_encoding(d_model, length):
    position = np.arange(length, dtype=np.float32)[:, None]
    div_term = np.exp(
        np.arange(0, d_model, 2, dtype=np.float32) * (-math.log(10000.0) / d_model))
    pe = np.zeros((length, d_model), dtype=np.float32)
    pe[:, 0::2] = np.sin(position * div_term)
    pe[:, 1::2] = np.cos(position * div_term)
    return jnp.asarray(pe)


def _sc_align_gather(enc_flat, align_flat, B, T, D):
    NW = _NUM_CORES * _NUM_SUBCORES
    halves = NW // B
    span = T // halves
    CH = 128
    n_scan = T // _LANES
    n_chunks = span // CH

    mesh = plsc.VectorSubcoreMesh(core_axis_name="c", subcore_axis_name="s")

    @functools.partial(
        pl.kernel,
        out_type=jax.ShapeDtypeStruct((B * T, D), jnp.float32),
        mesh=mesh,
        scratch_types=[
            pltpu.VMEM((8 + T,), jnp.int32),
            pltpu.VMEM((T,), jnp.int32),
            pltpu.VMEM((CH, D), jnp.float32),
            pltpu.VMEM((CH, D), jnp.float32),
            pltpu.SemaphoreType.DMA,
            pltpu.SemaphoreType.DMA,
        ],
        compiler_params=pltpu.CompilerParams(needs_layout_passes=False),
    )
    def sc_kernel(enc_hbm, align_hbm, out_hbm, align_v, inds_v,
                  rows_a, rows_b, sem_a, sem_b):
        wid = lax.axis_index("s") * _NUM_CORES + lax.axis_index("c")
        b = wid // halves
        h = wid % halves
        row0 = b * T
        pltpu.sync_copy(align_hbm.at[pl.ds(row0, T)], align_v.at[pl.ds(8, T)])
        iota = lax.iota(jnp.int32, _LANES)

        def scan_body(i, carry):
            t0 = i * _LANES
            a = align_v[pl.ds(8 + t0, _LANES)]
            prev = align_v[pl.ds(7 + t0, _LANES)]
            changed = jnp.logical_and(a != prev, (t0 + iota) > 0)
            flags = jnp.where(changed, jnp.int32(1), jnp.int32(0))
            vals = plsc.cumsum(flags) + carry
            inds_v[pl.ds(t0, _LANES)] = vals
            return jnp.max(vals)

        lax.fori_loop(0, n_scan, scan_body, jnp.int32(0) + row0)

        def gather_pair(j2, carry):
            off_a = h * span + (2 * j2) * CH
            off_b = off_a + CH
            da = pltpu.async_copy(enc_hbm.at[inds_v.at[pl.ds(off_a, CH)]],
                                  rows_a, sem_a)
            db = pltpu.async_copy(enc_hbm.at[inds_v.at[pl.ds(off_b, CH)]],
                                  rows_b, sem_b)
            da.wait()
            pltpu.sync_copy(rows_a, out_hbm.at[pl.ds(row0 + off_a, CH)])
            db.wait()
            pltpu.sync_copy(rows_b, out_hbm.at[pl.ds(row0 + off_b, CH)])
            return carry

        lax.fori_loop(0, n_chunks // 2, gather_pair, jnp.int32(0))

    return sc_kernel(enc_flat, align_flat)


def _tc_postnet(gathered, pe, pitch, beats_f, W2, wp, dvec, cvec, TT=1024):
    B, T, D = gathered.shape

    def body(g_ref, pe_ref, p_ref, bt_ref, w2_ref, wp_ref, dv_ref, cv_ref, o_ref):
        pe_t = pe_ref[...]
        x = g_ref[0] + pe_t
        acc = jnp.dot(x, w2_ref[...], preferred_element_type=jnp.float32)
        bt = bt_ref[0].astype(jnp.float32)
        o_ref[0] = (acc - pe_t + p_ref[0] * wp_ref[...]
                    + bt * dv_ref[...] + cv_ref[...])

    return pl.pallas_call(
        body,
        grid=(T // TT, B),
        in_specs=[
            pl.BlockSpec((1, TT, D), lambda t, b: (b, t, 0)),
            pl.BlockSpec((TT, D), lambda t, b: (t, 0)),
            pl.BlockSpec((1, TT, 1), lambda t, b: (b, t, 0)),
            pl.BlockSpec((1, TT, 1), lambda t, b: (b, t, 0)),
            pl.BlockSpec((D, D), lambda t, b: (0, 0)),
            pl.BlockSpec((1, D), lambda t, b: (0, 0)),
            pl.BlockSpec((1, D), lambda t, b: (0, 0)),
            pl.BlockSpec((1, D), lambda t, b: (0, 0)),
        ],
        out_specs=pl.BlockSpec((1, TT, D), lambda t, b: (b, t, 0)),
        out_shape=jax.ShapeDtypeStruct((B, T, D), jnp.float32),
    )(gathered, pe, pitch, beats_f, W2, wp, dvec, cvec)


def kernel(encoder_out, align_phone, text_phone, pitch, beats,
           W_pitch, b_pitch, W_pos, b_pos, emb_beats):
    del text_phone
    B, T, D = encoder_out.shape
    enc_flat = encoder_out.reshape(B * T, D)
    align_flat = align_phone.reshape(B * T)
    gathered = _sc_align_gather(enc_flat, align_flat, B, T, D).reshape(B, T, D)

    pe = _positional_encoding(D, T)
    W2 = W_pos.T + jnp.eye(D, dtype=W_pos.dtype)
    wp = W_pitch.reshape(1, D)
    dvec = (emb_beats[1] - emb_beats[0]).reshape(1, D)
    cvec = (b_pos + b_pitch + emb_beats[0]).reshape(1, D)
    return _tc_postnet(gathered, pe, pitch, beats, W2, wp, dvec, cvec)


# speedup vs baseline: 1.1135x; 1.1135x over previous
import functools
import math

import jax
import jax.numpy as jnp
import numpy as np
from jax import lax
from jax.experimental import pallas as pl
from jax.experimental.pallas import tpu as pltpu
from jax.experimental.pallas import tpu_sc as plsc

_NUM_CORES = 2
_NUM_SUBCORES = 16
_LANES = 16


def _positional_encoding(d_model, length):
    position = np.arange(length, dtype=np.float32)[:, None]
    div_term = np.exp(
        np.arange(0, d_model, 2, dtype=np.float32) * (-math.log(10000.0) / d_model))
    pe = np.zeros((length, d_model), dtype=np.float32)
    pe[:, 0::2] = np.sin(position * div_term)
    pe[:, 1::2] = np.cos(position * div_term)
    return jnp.asarray(pe)


def _sc_align_gather(enc_flat, align_flat, B, T, D):
    NW = _NUM_CORES * _NUM_SUBCORES
    halves = NW // B
    span = T // halves
    CH = 128
    n_scan = T // _LANES
    n_chunks = span // CH

    mesh = plsc.VectorSubcoreMesh(core_axis_name="c", subcore_axis_name="s")

    @functools.partial(
        pl.kernel,
        out_type=jax.ShapeDtypeStruct((B * T, D), jnp.float32),
        mesh=mesh,
        scratch_types=[
            pltpu.VMEM((8 + T,), jnp.int32),
            pltpu.VMEM((T,), jnp.int32),
            pltpu.VMEM((CH, D), jnp.float32),
            pltpu.VMEM((CH, D), jnp.float32),
            pltpu.SemaphoreType.DMA,
            pltpu.SemaphoreType.DMA,
        ],
        compiler_params=pltpu.CompilerParams(needs_layout_passes=False),
    )
    def sc_kernel(enc_hbm, align_hbm, out_hbm, align_v, inds_v,
                  rows_a, rows_b, sem_a, sem_b):
        wid = lax.axis_index("s") * _NUM_CORES + lax.axis_index("c")
        b = wid // halves
        h = wid % halves
        row0 = b * T
        pltpu.sync_copy(align_hbm.at[pl.ds(row0, T)], align_v.at[pl.ds(8, T)])
        iota = lax.iota(jnp.int32, _LANES)

        def scan_body(i, carry):
            t0 = i * _LANES
            a = align_v[pl.ds(8 + t0, _LANES)]
            prev = align_v[pl.ds(7 + t0, _LANES)]
            changed = jnp.logical_and(a != prev, (t0 + iota) > 0)
            flags = jnp.where(changed, jnp.int32(1), jnp.int32(0))
            vals = plsc.cumsum(flags) + carry
            inds_v[pl.ds(t0, _LANES)] = vals
            return jnp.max(vals)

        lax.fori_loop(0, n_scan, scan_body, jnp.int32(0) + row0)

        def gather_pair(j2, carry):
            off_a = h * span + (2 * j2) * CH
            off_b = off_a + CH
            da = pltpu.async_copy(enc_hbm.at[inds_v.at[pl.ds(off_a, CH)]],
                                  rows_a, sem_a)
            db = pltpu.async_copy(enc_hbm.at[inds_v.at[pl.ds(off_b, CH)]],
                                  rows_b, sem_b)
            da.wait()
            pltpu.sync_copy(rows_a, out_hbm.at[pl.ds(row0 + off_a, CH)])
            db.wait()
            pltpu.sync_copy(rows_b, out_hbm.at[pl.ds(row0 + off_b, CH)])
            return carry

        lax.fori_loop(0, n_chunks // 2, gather_pair, jnp.int32(0))

    return sc_kernel(enc_flat, align_flat)


def _tc_postnet(gathered, pe, pitch, beats_f, W2, wp, dvec, cvec, TT=4096):
    B, T, D = gathered.shape

    def body(g_ref, pe_ref, p_ref, bt_ref, w2_ref, wp_ref, dv_ref, cv_ref, o_ref):
        pe_t = pe_ref[...]
        x = g_ref[0] + pe_t
        acc = jnp.dot(x, w2_ref[...], preferred_element_type=jnp.float32)
        bt = bt_ref[0].astype(jnp.float32)
        o_ref[0] = (acc - pe_t + p_ref[0] * wp_ref[...]
                    + bt * dv_ref[...] + cv_ref[...])

    return pl.pallas_call(
        body,
        grid=(T // TT, B),
        in_specs=[
            pl.BlockSpec((1, TT, D), lambda t, b: (b, t, 0)),
            pl.BlockSpec((TT, D), lambda t, b: (t, 0)),
            pl.BlockSpec((1, TT, 1), lambda t, b: (b, t, 0)),
            pl.BlockSpec((1, TT, 1), lambda t, b: (b, t, 0)),
            pl.BlockSpec((D, D), lambda t, b: (0, 0)),
            pl.BlockSpec((1, D), lambda t, b: (0, 0)),
            pl.BlockSpec((1, D), lambda t, b: (0, 0)),
            pl.BlockSpec((1, D), lambda t, b: (0, 0)),
        ],
        out_specs=pl.BlockSpec((1, TT, D), lambda t, b: (b, t, 0)),
        out_shape=jax.ShapeDtypeStruct((B, T, D), jnp.float32),
    )(gathered, pe, pitch, beats_f, W2, wp, dvec, cvec)


def kernel(encoder_out, align_phone, text_phone, pitch, beats,
           W_pitch, b_pitch, W_pos, b_pos, emb_beats):
    del text_phone
    B, T, D = encoder_out.shape
    enc_flat = encoder_out.reshape(B * T, D)
    align_flat = align_phone.reshape(B * T)
    gathered = _sc_align_gather(enc_flat, align_flat, B, T, D).reshape(B, T, D)

    pe = _positional_encoding(D, T)
    W2 = W_pos.T + jnp.eye(D, dtype=W_pos.dtype)
    wp = W_pitch.reshape(1, D)
    dvec = (emb_beats[1] - emb_beats[0]).reshape(1, D)
    cvec = (b_pos + b_pitch + emb_beats[0]).reshape(1, D)
    return _tc_postnet(gathered, pe, pitch, beats, W2, wp, dvec, cvec)
